# baseline (speedup 1.0000x reference)
import functools

import jax
import jax.numpy as jnp
from jax.experimental import pallas as pl
from jax.experimental.pallas import tpu as pltpu

_VMEM_LIMIT = 48 * 1024 * 1024
_EPS = 1e-5


def _round_up(a, b):
    return (a + b - 1) // b * b


def _scale_shift(st_ref, g_ref, be_ref, m):
    st = jnp.sum(st_ref[...], axis=0)
    mean = st[0:1] / m
    var = jnp.maximum(st[1:2] / m - mean * mean, 0.0)
    scale = g_ref[...] / jnp.sqrt(var + _EPS)
    shift = be_ref[...] - mean * scale
    return scale, shift


def _k1_conv1(x_ref, w1_ref, b1_ref, y1_ref, st1_ref):
    xb = x_ref[0].astype(jnp.bfloat16)
    y1 = jax.lax.dot_general(xb, w1_ref[...], (((0,), (0,)), ((), ())),
                             preferred_element_type=jnp.float32)
    y1 = y1 + b1_ref[...]
    y1_ref[0] = y1.astype(jnp.bfloat16)
    st1_ref[0, 0:1, :] = jnp.sum(y1, axis=0, keepdims=True)
    st1_ref[0, 1:2, :] = jnp.sum(y1 * y1, axis=0, keepdims=True)


def _k2_conv2_ds(ph_ref, w2_ref, b2_ref, xs_ref, wd_ref, bd_ref,
                 y2_ref, st2_ref, yd_ref, std_ref, *, ho, wo, wo_pad, hps):
    ph = ph_ref[0]
    c = ph.shape[-1]
    wins = []
    for di in range(3):
        for dj in range(3):
            v = (di % 2) * 3 + dj
            r0 = v * hps + di // 2
            wins.append(ph[r0:r0 + ho].reshape(ho * wo_pad, c))
    xw = jnp.concatenate(wins, axis=1)
    y2 = jnp.dot(xw, w2_ref[...],
                 preferred_element_type=jnp.float32) + b2_ref[...]
    y2_ref[0] = y2.astype(jnp.bfloat16)
    row = jax.lax.broadcasted_iota(jnp.int32, (y2.shape[0], 1), 0)
    valid = row % wo_pad < wo
    y2v = jnp.where(valid, y2, 0.0)
    st2_ref[0, 0:1, :] = jnp.sum(y2v, axis=0, keepdims=True)
    st2_ref[0, 1:2, :] = jnp.sum(y2v * y2v, axis=0, keepdims=True)

    yd = jax.lax.dot_general(xs_ref[0], wd_ref[...], (((0,), (0,)), ((), ())),
                             preferred_element_type=jnp.float32)
    yd = yd + bd_ref[...]
    yd_ref[0] = yd.astype(jnp.bfloat16)
    ydv = jnp.where(valid, yd, 0.0)
    std_ref[0, 0:1, :] = jnp.sum(ydv, axis=0, keepdims=True)
    std_ref[0, 1:2, :] = jnp.sum(ydv * ydv, axis=0, keepdims=True)


def _k3_conv3(y2_ref, st2_ref, g2_ref, be2_ref, w3_ref, b3_ref,
              y3_ref, st3_ref, *, m2, wo_pad, wo):
    s2, h2 = _scale_shift(st2_ref, g2_ref, be2_ref, m2)
    t = y2_ref[...].astype(jnp.float32) * s2 + h2
    a2 = jnp.where(t >= 0, t, 0.02 * t).astype(jnp.bfloat16)
    y3 = jnp.dot(a2, w3_ref[...],
                 preferred_element_type=jnp.float32) + b3_ref[...]
    y3_ref[...] = y3.astype(jnp.bfloat16)
    row = jax.lax.broadcasted_iota(jnp.int32, (y3.shape[0], 1), 0)
    y3v = jnp.where(row % wo_pad < wo, y3, 0.0)
    st3_ref[0, 0:1, :] = jnp.sum(y3v, axis=0, keepdims=True)
    st3_ref[0, 1:2, :] = jnp.sum(y3v * y3v, axis=0, keepdims=True)


def _k4_residual(y3_ref, st3_ref, g3_ref, be3_ref, yd_ref, std_ref,
                 gd_ref, bed_ref, o_ref, *, m2):
    s3, h3 = _scale_shift(st3_ref, g3_ref, be3_ref, m2)
    sd, hd = _scale_shift(std_ref, gd_ref, bed_ref, m2)
    z = y3_ref[...].astype(jnp.float32) * s3 + h3
    sk = yd_ref[...].astype(jnp.float32) * sd + hd
    y = z + sk
    o_ref[...] = jnp.where(y >= 0, y, 0.01 * y)


def kernel(x, w1, b1, g1, be1, w2, b2, g2, be2, w3, b3, g3, be3,
           wd, bd, gd, bed):
    n, cin, h, w = x.shape
    cout = w1.shape[1]
    c4 = w3.shape[1]
    stride = 2
    ho = (h + 2 - 3) // stride + 1
    wo = (w + 2 - 3) // stride + 1
    wo_pad = _round_up(wo, 8)
    mo = ho * wo_pad
    hw = h * w
    bf = jnp.bfloat16

    xr = x.reshape(n, cin, hw)
    y1, st1 = pl.pallas_call(
        _k1_conv1,
        out_shape=(jax.ShapeDtypeStruct((n, hw, cout), bf),
                   jax.ShapeDtypeStruct((n, 2, cout), jnp.float32)),
        grid=(n,),
        in_specs=[pl.BlockSpec((1, cin, hw), lambda i: (i, 0, 0)),
                  pl.BlockSpec((cin, cout), lambda i: (0, 0)),
                  pl.BlockSpec((1, cout), lambda i: (0, 0))],
        out_specs=(pl.BlockSpec((1, hw, cout), lambda i: (i, 0, 0)),
                   pl.BlockSpec((1, 2, cout), lambda i: (i, 0, 0))),
        compiler_params=pltpu.CompilerParams(
            dimension_semantics=("parallel",),
            vmem_limit_bytes=_VMEM_LIMIT),
    )(xr, w1.astype(bf), b1.reshape(1, cout))

    m1 = n * hw
    st1s = jnp.sum(st1, axis=0)
    mean1 = st1s[0] / m1
    var1 = jnp.maximum(st1s[1] / m1 - mean1 * mean1, 0.0)
    s1 = g1 / jnp.sqrt(var1 + _EPS)
    h1 = be1 - mean1 * s1
    a1 = y1.astype(jnp.float32) * s1 + h1
    a1 = jnp.where(a1 >= 0, a1, 0.02 * a1).astype(bf)
    a1 = a1.reshape(n, h, w, cout)
    w_pad = 2 * wo_pad + 2
    a1p = jnp.pad(a1, ((0, 0), (1, 1), (1, w_pad - w - 1), (0, 0)))
    hps = (h + 2) // 2
    variants = [a1p[:, pi::2, dj:dj + 2 * wo_pad:2, :]
                for pi in range(2) for dj in range(3)]
    ph = jnp.stack(variants, axis=1).reshape(n, 6 * hps, wo_pad, cout)

    xs = x[:, :, ::stride, ::stride].astype(bf)
    xs = jnp.pad(xs, ((0, 0), (0, 0), (0, 0), (0, wo_pad - wo)))
    xs = xs.reshape(n, cin, mo)

    y2, st2, yd, std = pl.pallas_call(
        functools.partial(_k2_conv2_ds, ho=ho, wo=wo, wo_pad=wo_pad, hps=hps),
        out_shape=(jax.ShapeDtypeStruct((n, mo, cout), bf),
                   jax.ShapeDtypeStruct((n, 2, cout), jnp.float32),
                   jax.ShapeDtypeStruct((n, mo, c4), bf),
                   jax.ShapeDtypeStruct((n, 2, c4), jnp.float32)),
        grid=(n,),
        in_specs=[pl.BlockSpec((1, 6 * hps, wo_pad, cout),
                               lambda i: (i, 0, 0, 0)),
                  pl.BlockSpec((9 * cout, cout), lambda i: (0, 0)),
                  pl.BlockSpec((1, cout), lambda i: (0, 0)),
                  pl.BlockSpec((1, cin, mo), lambda i: (i, 0, 0)),
                  pl.BlockSpec((cin, c4), lambda i: (0, 0)),
                  pl.BlockSpec((1, c4), lambda i: (0, 0))],
        out_specs=(pl.BlockSpec((1, mo, cout), lambda i: (i, 0, 0)),
                   pl.BlockSpec((1, 2, cout), lambda i: (i, 0, 0)),
                   pl.BlockSpec((1, mo, c4), lambda i: (i, 0, 0)),
                   pl.BlockSpec((1, 2, c4), lambda i: (i, 0, 0))),
        compiler_params=pltpu.CompilerParams(
            dimension_semantics=("parallel",),
            vmem_limit_bytes=_VMEM_LIMIT),
    )(ph, w2.reshape(9 * cout, cout).astype(bf), b2.reshape(1, cout),
      xs, wd.astype(bf), bd.reshape(1, c4))

    m2 = n * ho * wo

    mtot = n * mo
    tm = 1024 if mtot % 1024 == 0 else mo
    nt = mtot // tm
    y3, st3 = pl.pallas_call(
        functools.partial(_k3_conv3, m2=m2, wo_pad=wo_pad, wo=wo),
        out_shape=(jax.ShapeDtypeStruct((mtot, c4), bf),
                   jax.ShapeDtypeStruct((nt, 2, c4), jnp.float32)),
        grid=(nt,),
        in_specs=[pl.BlockSpec((tm, cout), lambda i: (i, 0)),
                  pl.BlockSpec((n, 2, cout), lambda i: (0, 0, 0)),
                  pl.BlockSpec((1, cout), lambda i: (0, 0)),
                  pl.BlockSpec((1, cout), lambda i: (0, 0)),
                  pl.BlockSpec((cout, c4), lambda i: (0, 0)),
                  pl.BlockSpec((1, c4), lambda i: (0, 0))],
        out_specs=(pl.BlockSpec((tm, c4), lambda i: (i, 0)),
                   pl.BlockSpec((1, 2, c4), lambda i: (i, 0, 0))),
        compiler_params=pltpu.CompilerParams(
            dimension_semantics=("parallel",),
            vmem_limit_bytes=_VMEM_LIMIT),
    )(y2.reshape(mtot, cout), st2, g2.reshape(1, cout), be2.reshape(1, cout),
      w3.astype(bf), b3.reshape(1, c4))

    out = pl.pallas_call(
        functools.partial(_k4_residual, m2=m2),
        out_shape=jax.ShapeDtypeStruct((mtot, c4), jnp.float32),
        grid=(nt,),
        in_specs=[pl.BlockSpec((tm, c4), lambda i: (i, 0)),
                  pl.BlockSpec((nt, 2, c4), lambda i: (0, 0, 0)),
                  pl.BlockSpec((1, c4), lambda i: (0, 0)),
                  pl.BlockSpec((1, c4), lambda i: (0, 0)),
                  pl.BlockSpec((tm, c4), lambda i: (i, 0)),
                  pl.BlockSpec((n, 2, c4), lambda i: (0, 0, 0)),
                  pl.BlockSpec((1, c4), lambda i: (0, 0)),
                  pl.BlockSpec((1, c4), lambda i: (0, 0))],
        out_specs=pl.BlockSpec((tm, c4), lambda i: (i, 0)),
        compiler_params=pltpu.CompilerParams(
            dimension_semantics=("parallel",),
            vmem_limit_bytes=_VMEM_LIMIT),
    )(y3, st3, g3.reshape(1, c4), be3.reshape(1, c4),
      yd.reshape(mtot, c4), std, gd.reshape(1, c4), bed.reshape(1, c4))

    out = out.reshape(n, ho, wo_pad, c4)[:, :, :wo, :]
    return jnp.transpose(out, (0, 3, 1, 2))


# speedup vs baseline: 6.5472x; 6.5472x over previous
import functools

import jax
import jax.numpy as jnp
from jax.experimental import pallas as pl
from jax.experimental.pallas import tpu as pltpu

_VMEM_LIMIT = 48 * 1024 * 1024
_EPS = 1e-5


def _round_up(a, b):
    return (a + b - 1) // b * b


def _scale_shift(st_ref, g_ref, be_ref, m):
    st = jnp.sum(st_ref[...], axis=0)
    mean = st[0:1] / m
    var = jnp.maximum(st[1:2] / m - mean * mean, 0.0)
    scale = g_ref[...] / jnp.sqrt(var + _EPS)
    shift = be_ref[...] - mean * scale
    return scale, shift


def _k1_conv1(x_ref, w1_ref, b1_ref, y1_ref, st1_ref):
    xb = x_ref[...].astype(jnp.bfloat16)
    y1 = jnp.dot(xb, w1_ref[...], preferred_element_type=jnp.float32)
    y1 = y1 + b1_ref[...]
    y1_ref[...] = y1.astype(jnp.bfloat16)
    st1_ref[0, 0:1, :] = jnp.sum(y1, axis=0, keepdims=True)
    st1_ref[0, 1:2, :] = jnp.sum(y1 * y1, axis=0, keepdims=True)


def _k2a_pad(y1_ref, st1_ref, g1_ref, be1_ref, p_ref, *, m1, h, w, hp, wp):
    s1, h1 = _scale_shift(st1_ref, g1_ref, be1_ref, m1)
    a = y1_ref[0].astype(jnp.float32) * s1 + h1
    a = jnp.where(a >= 0, a, 0.02 * a).astype(jnp.bfloat16)
    p_ref[0] = jnp.pad(a, ((1, hp - h - 1), (1, wp - w - 1), (0, 0)))


def _k2b_conv2_ds(p00_ref, p10_ref, p01_ref, p11_ref, xs_ref, w2_ref, b2_ref,
                  wd_ref, bd_ref, y2_ref, st2_ref, yd_ref, std_ref, *,
                  ho, wo, wo_pad):
    c = w2_ref.shape[1]
    phases = {(0, 0): p00_ref[0, :, 0, :, :],
              (1, 0): p10_ref[0, :, 0, :, :],
              (0, 1): p01_ref[0, :, 0, :, :],
              (1, 1): p11_ref[0, :, 0, :, :]}
    wins = []
    for di in range(3):
        for dj in range(3):
            v = phases[(di % 2, dj % 2)]
            if dj == 2:
                v = jnp.roll(v, -1, axis=1)
            wins.append(v[di // 2:di // 2 + ho].reshape(ho * wo_pad, c))
    xw = jnp.concatenate(wins, axis=1)
    y2 = jnp.dot(xw, w2_ref[...],
                 preferred_element_type=jnp.float32) + b2_ref[...]
    y2_ref[0] = y2.astype(jnp.bfloat16)
    row = jax.lax.broadcasted_iota(jnp.int32, (y2.shape[0], 1), 0)
    valid = row % wo_pad < wo
    y2v = jnp.where(valid, y2, 0.0)
    st2_ref[0, 0:1, :] = jnp.sum(y2v, axis=0, keepdims=True)
    st2_ref[0, 1:2, :] = jnp.sum(y2v * y2v, axis=0, keepdims=True)

    cin = xs_ref.shape[4]
    xs = xs_ref[0, :, 0, :, :]
    xs = jnp.pad(xs, ((0, 0), (0, wo_pad - wo), (0, 0)))
    xs = xs.reshape(ho * wo_pad, cin).astype(jnp.bfloat16)
    yd = jnp.dot(xs, wd_ref[...],
                 preferred_element_type=jnp.float32) + bd_ref[...]
    yd_ref[0] = yd.astype(jnp.bfloat16)
    ydv = jnp.where(valid, yd, 0.0)
    std_ref[0, 0:1, :] = jnp.sum(ydv, axis=0, keepdims=True)
    std_ref[0, 1:2, :] = jnp.sum(ydv * ydv, axis=0, keepdims=True)


def _k3_conv3(y2_ref, st2_ref, g2_ref, be2_ref, w3_ref, b3_ref,
              y3_ref, st3_ref, *, m2, wo_pad, wo):
    s2, h2 = _scale_shift(st2_ref, g2_ref, be2_ref, m2)
    t = y2_ref[...].astype(jnp.float32) * s2 + h2
    a2 = jnp.where(t >= 0, t, 0.02 * t).astype(jnp.bfloat16)
    y3 = jnp.dot(a2, w3_ref[...],
                 preferred_element_type=jnp.float32) + b3_ref[...]
    y3_ref[...] = y3.astype(jnp.bfloat16)
    row = jax.lax.broadcasted_iota(jnp.int32, (y3.shape[0], 1), 0)
    y3v = jnp.where(row % wo_pad < wo, y3, 0.0)
    st3_ref[0, 0:1, :] = jnp.sum(y3v, axis=0, keepdims=True)
    st3_ref[0, 1:2, :] = jnp.sum(y3v * y3v, axis=0, keepdims=True)


def _k4_residual(y3_ref, st3_ref, g3_ref, be3_ref, yd_ref, std_ref,
                 gd_ref, bed_ref, o_ref, *, m2, ho, wo, wo_pad):
    s3, h3 = _scale_shift(st3_ref, g3_ref, be3_ref, m2)
    sd, hd = _scale_shift(std_ref, gd_ref, bed_ref, m2)
    z = y3_ref[0].astype(jnp.float32) * s3 + h3
    sk = yd_ref[0].astype(jnp.float32) * sd + hd
    y = z + sk
    y = jnp.where(y >= 0, y, 0.01 * y)
    c4 = y.shape[-1]
    o_ref[0] = y.reshape(ho, wo_pad, c4)[:, :wo, :]


def kernel(x, w1, b1, g1, be1, w2, b2, g2, be2, w3, b3, g3, be3,
           wd, bd, gd, bed):
    n, cin, h, w = x.shape
    cout = w1.shape[1]
    c4 = w3.shape[1]
    ho = (h + 2 - 3) // 2 + 1
    wo = (w + 2 - 3) // 2 + 1
    wo_pad = _round_up(wo, 8)
    mo = ho * wo_pad
    hw = h * w
    bf = jnp.bfloat16
    hp = _round_up(h + 2, 16)
    wp = _round_up(w + 2, 16)

    x_nhwc = jnp.transpose(x, (0, 2, 3, 1))
    x_flat = x_nhwc.reshape(n * hw, cin)

    m1 = n * hw
    tm1 = 1568 if m1 % 1568 == 0 else hw
    nt1 = m1 // tm1
    y1, st1 = pl.pallas_call(
        _k1_conv1,
        out_shape=(jax.ShapeDtypeStruct((m1, cout), bf),
                   jax.ShapeDtypeStruct((nt1, 2, cout), jnp.float32)),
        grid=(nt1,),
        in_specs=[pl.BlockSpec((tm1, cin), lambda i: (i, 0)),
                  pl.BlockSpec((cin, cout), lambda i: (0, 0)),
                  pl.BlockSpec((1, cout), lambda i: (0, 0))],
        out_specs=(pl.BlockSpec((tm1, cout), lambda i: (i, 0)),
                   pl.BlockSpec((1, 2, cout), lambda i: (i, 0, 0))),
        compiler_params=pltpu.CompilerParams(
            dimension_semantics=("parallel",),
            vmem_limit_bytes=_VMEM_LIMIT),
    )(x_flat, w1.astype(bf), b1.reshape(1, cout))

    p = pl.pallas_call(
        functools.partial(_k2a_pad, m1=m1, h=h, w=w, hp=hp, wp=wp),
        out_shape=jax.ShapeDtypeStruct((n, hp, wp, cout), bf),
        grid=(n,),
        in_specs=[pl.BlockSpec((1, h, w, cout), lambda i: (i, 0, 0, 0)),
                  pl.BlockSpec((nt1, 2, cout), lambda i: (0, 0, 0)),
                  pl.BlockSpec((1, cout), lambda i: (0, 0)),
                  pl.BlockSpec((1, cout), lambda i: (0, 0))],
        out_specs=pl.BlockSpec((1, hp, wp, cout), lambda i: (i, 0, 0, 0)),
        compiler_params=pltpu.CompilerParams(
            dimension_semantics=("parallel",),
            vmem_limit_bytes=_VMEM_LIMIT),
    )(y1.reshape(n, h, w, cout), st1, g1.reshape(1, cout),
      be1.reshape(1, cout))

    pv = p.reshape(n, hp // 2, 2, wp // 2, 2 * cout)
    xv = x_nhwc.reshape(n, h // 2, 2, w // 2, 2 * cin)

    def _phase(r, s):
        return pl.BlockSpec((1, hp // 2, 1, wp // 2, cout),
                            lambda i, _r=r, _s=s: (i, 0, _r, 0, _s))

    y2, st2, yd, std = pl.pallas_call(
        functools.partial(_k2b_conv2_ds, ho=ho, wo=wo, wo_pad=wo_pad),
        out_shape=(jax.ShapeDtypeStruct((n, mo, cout), bf),
                   jax.ShapeDtypeStruct((n, 2, cout), jnp.float32),
                   jax.ShapeDtypeStruct((n, mo, c4), bf),
                   jax.ShapeDtypeStruct((n, 2, c4), jnp.float32)),
        grid=(n,),
        in_specs=[_phase(0, 0), _phase(1, 0), _phase(0, 1), _phase(1, 1),
                  pl.BlockSpec((1, h // 2, 1, w // 2, cin),
                               lambda i: (i, 0, 0, 0, 0)),
                  pl.BlockSpec((9 * cout, cout), lambda i: (0, 0)),
                  pl.BlockSpec((1, cout), lambda i: (0, 0)),
                  pl.BlockSpec((cin, c4), lambda i: (0, 0)),
                  pl.BlockSpec((1, c4), lambda i: (0, 0))],
        out_specs=(pl.BlockSpec((1, mo, cout), lambda i: (i, 0, 0)),
                   pl.BlockSpec((1, 2, cout), lambda i: (i, 0, 0)),
                   pl.BlockSpec((1, mo, c4), lambda i: (i, 0, 0)),
                   pl.BlockSpec((1, 2, c4), lambda i: (i, 0, 0))),
        compiler_params=pltpu.CompilerParams(
            dimension_semantics=("parallel",),
            vmem_limit_bytes=_VMEM_LIMIT),
    )(pv, pv, pv, pv, xv, w2.reshape(9 * cout, cout).astype(bf),
      b2.reshape(1, cout), wd.astype(bf), bd.reshape(1, c4))

    m2 = n * ho * wo

    mtot = n * mo
    tm = 1024 if mtot % 1024 == 0 else mo
    nt = mtot // tm
    y3, st3 = pl.pallas_call(
        functools.partial(_k3_conv3, m2=m2, wo_pad=wo_pad, wo=wo),
        out_shape=(jax.ShapeDtypeStruct((mtot, c4), bf),
                   jax.ShapeDtypeStruct((nt, 2, c4), jnp.float32)),
        grid=(nt,),
        in_specs=[pl.BlockSpec((tm, cout), lambda i: (i, 0)),
                  pl.BlockSpec((n, 2, cout), lambda i: (0, 0, 0)),
                  pl.BlockSpec((1, cout), lambda i: (0, 0)),
                  pl.BlockSpec((1, cout), lambda i: (0, 0)),
                  pl.BlockSpec((cout, c4), lambda i: (0, 0)),
                  pl.BlockSpec((1, c4), lambda i: (0, 0))],
        out_specs=(pl.BlockSpec((tm, c4), lambda i: (i, 0)),
                   pl.BlockSpec((1, 2, c4), lambda i: (i, 0, 0))),
        compiler_params=pltpu.CompilerParams(
            dimension_semantics=("parallel",),
            vmem_limit_bytes=_VMEM_LIMIT),
    )(y2.reshape(mtot, cout), st2, g2.reshape(1, cout), be2.reshape(1, cout),
      w3.astype(bf), b3.reshape(1, c4))

    out = pl.pallas_call(
        functools.partial(_k4_residual, m2=m2, ho=ho, wo=wo, wo_pad=wo_pad),
        out_shape=jax.ShapeDtypeStruct((n, ho, wo, c4), jnp.float32),
        grid=(n,),
        in_specs=[pl.BlockSpec((1, mo, c4), lambda i: (i, 0, 0)),
                  pl.BlockSpec((nt, 2, c4), lambda i: (0, 0, 0)),
                  pl.BlockSpec((1, c4), lambda i: (0, 0)),
                  pl.BlockSpec((1, c4), lambda i: (0, 0)),
                  pl.BlockSpec((1, mo, c4), lambda i: (i, 0, 0)),
                  pl.BlockSpec((n, 2, c4), lambda i: (0, 0, 0)),
                  pl.BlockSpec((1, c4), lambda i: (0, 0)),
                  pl.BlockSpec((1, c4), lambda i: (0, 0))],
        out_specs=pl.BlockSpec((1, ho, wo, c4), lambda i: (i, 0, 0, 0)),
        compiler_params=pltpu.CompilerParams(
            dimension_semantics=("parallel",),
            vmem_limit_bytes=_VMEM_LIMIT),
    )(y3.reshape(n, mo, c4), st3, g3.reshape(1, c4), be3.reshape(1, c4),
      yd, std, gd.reshape(1, c4), bed.reshape(1, c4))

    return jnp.transpose(out, (0, 3, 1, 2))


# speedup vs baseline: 7.2833x; 1.1124x over previous
import functools

import jax
import jax.numpy as jnp
from jax.experimental import pallas as pl
from jax.experimental.pallas import tpu as pltpu

_VMEM_LIMIT = 48 * 1024 * 1024
_EPS = 1e-5


def _round_up(a, b):
    return (a + b - 1) // b * b


def _scale_shift(st_ref, g_ref, be_ref, m):
    st = jnp.sum(st_ref[...], axis=0)
    mean = st[0:1] / m
    var = jnp.maximum(st[1:2] / m - mean * mean, 0.0)
    scale = g_ref[...] / jnp.sqrt(var + _EPS)
    shift = be_ref[...] - mean * scale
    return scale, shift


def _k1_conv1(x_ref, w1_ref, b1_ref, y1_ref, st1_ref):
    xb = x_ref[...].astype(jnp.bfloat16)
    y1 = jnp.dot(xb, w1_ref[...].astype(jnp.bfloat16),
                 preferred_element_type=jnp.float32)
    y1 = y1 + b1_ref[...]
    y1_ref[...] = y1.astype(jnp.bfloat16)
    st1_ref[0, 0:1, :] = jnp.sum(y1, axis=0, keepdims=True)
    st1_ref[0, 1:2, :] = jnp.sum(y1 * y1, axis=0, keepdims=True)


def _k2a_pad(y1_ref, st1_ref, g1_ref, be1_ref, p_ref, *, m1, h, w, hp, wp):
    s1, h1 = _scale_shift(st1_ref, g1_ref, be1_ref, m1)
    a = y1_ref[0].astype(jnp.float32) * s1 + h1
    a = jnp.where(a >= 0, a, 0.02 * a).astype(jnp.bfloat16)
    p_ref[0] = jnp.pad(a, ((1, hp - h - 1), (1, wp - w - 1), (0, 0)))


def _k2b_conv2_ds(p00_ref, p10_ref, p01_ref, p11_ref, xs_ref, w2_ref, b2_ref,
                  wd_ref, bd_ref, y2_ref, st2_ref, yd_ref, std_ref, *,
                  ho, wo, wo_pad):
    c = w2_ref.shape[1]
    phases = {(0, 0): p00_ref[0, :, 0, :, :],
              (1, 0): p10_ref[0, :, 0, :, :],
              (0, 1): p01_ref[0, :, 0, :, :],
              (1, 1): p11_ref[0, :, 0, :, :]}
    wins = []
    for di in range(3):
        for dj in range(3):
            v = phases[(di % 2, dj % 2)]
            if dj == 2:
                v = jnp.roll(v, -1, axis=1)
            wins.append(v[di // 2:di // 2 + ho].reshape(ho * wo_pad, c))
    xw = jnp.concatenate(wins, axis=1)
    y2 = jnp.dot(xw, w2_ref[...].astype(jnp.bfloat16),
                 preferred_element_type=jnp.float32) + b2_ref[...]
    y2 = y2.reshape(ho, wo_pad, c)[:, :wo, :].reshape(ho * wo, c)
    y2_ref[...] = y2.astype(jnp.bfloat16)
    st2_ref[0, 0:1, :] = jnp.sum(y2, axis=0, keepdims=True)
    st2_ref[0, 1:2, :] = jnp.sum(y2 * y2, axis=0, keepdims=True)

    cin = xs_ref.shape[4]
    xs = xs_ref[0, :, 0, :, :]
    xs = xs.astype(jnp.bfloat16).reshape(ho * wo, cin)
    yd = jnp.dot(xs, wd_ref[...].astype(jnp.bfloat16),
                 preferred_element_type=jnp.float32) + bd_ref[...]
    yd_ref[...] = yd.astype(jnp.bfloat16)
    std_ref[0, 0:1, :] = jnp.sum(yd, axis=0, keepdims=True)
    std_ref[0, 1:2, :] = jnp.sum(yd * yd, axis=0, keepdims=True)


def _k3_conv3(y2_ref, st2_ref, g2_ref, be2_ref, w3_ref, b3_ref,
              y3_ref, st3_ref, *, m2):
    s2, h2 = _scale_shift(st2_ref, g2_ref, be2_ref, m2)
    t = y2_ref[...].astype(jnp.float32) * s2 + h2
    a2 = jnp.where(t >= 0, t, 0.02 * t).astype(jnp.bfloat16)
    y3 = jnp.dot(a2, w3_ref[...].astype(jnp.bfloat16),
                 preferred_element_type=jnp.float32) + b3_ref[...]
    y3_ref[...] = y3.astype(jnp.bfloat16)
    st3_ref[0, 0:1, :] = jnp.sum(y3, axis=0, keepdims=True)
    st3_ref[0, 1:2, :] = jnp.sum(y3 * y3, axis=0, keepdims=True)


def _k4_residual(y3_ref, st3_ref, g3_ref, be3_ref, yd_ref, std_ref,
                 gd_ref, bed_ref, o_ref, *, m2):
    s3, h3 = _scale_shift(st3_ref, g3_ref, be3_ref, m2)
    sd, hd = _scale_shift(std_ref, gd_ref, bed_ref, m2)
    z = y3_ref[...].astype(jnp.float32) * s3 + h3
    sk = yd_ref[...].astype(jnp.float32) * sd + hd
    y = z + sk
    o_ref[...] = jnp.where(y >= 0, y, 0.01 * y)


def kernel(x, w1, b1, g1, be1, w2, b2, g2, be2, w3, b3, g3, be3,
           wd, bd, gd, bed):
    n, cin, h, w = x.shape
    cout = w1.shape[1]
    c4 = w3.shape[1]
    ho = (h + 2 - 3) // 2 + 1
    wo = (w + 2 - 3) // 2 + 1
    wo_pad = _round_up(wo, 8)
    hw = h * w
    sp = ho * wo
    bf = jnp.bfloat16
    hp = _round_up(h + 2, 16)
    wp = _round_up(w + 2, 16)

    x_nhwc = jnp.transpose(x, (0, 2, 3, 1))
    x_flat = x_nhwc.reshape(n * hw, cin)

    m1 = n * hw
    tm1 = 1568 if m1 % 1568 == 0 else hw
    nt1 = m1 // tm1
    y1, st1 = pl.pallas_call(
        _k1_conv1,
        out_shape=(jax.ShapeDtypeStruct((m1, cout), bf),
                   jax.ShapeDtypeStruct((nt1, 2, cout), jnp.float32)),
        grid=(nt1,),
        in_specs=[pl.BlockSpec((tm1, cin), lambda i: (i, 0)),
                  pl.BlockSpec((cin, cout), lambda i: (0, 0)),
                  pl.BlockSpec((1, cout), lambda i: (0, 0))],
        out_specs=(pl.BlockSpec((tm1, cout), lambda i: (i, 0)),
                   pl.BlockSpec((1, 2, cout), lambda i: (i, 0, 0))),
        compiler_params=pltpu.CompilerParams(
            dimension_semantics=("parallel",),
            vmem_limit_bytes=_VMEM_LIMIT),
    )(x_flat, w1, b1.reshape(1, cout))

    p = pl.pallas_call(
        functools.partial(_k2a_pad, m1=m1, h=h, w=w, hp=hp, wp=wp),
        out_shape=jax.ShapeDtypeStruct((n, hp, wp, cout), bf),
        grid=(n,),
        in_specs=[pl.BlockSpec((1, h, w, cout), lambda i: (i, 0, 0, 0)),
                  pl.BlockSpec((nt1, 2, cout), lambda i: (0, 0, 0)),
                  pl.BlockSpec((1, cout), lambda i: (0, 0)),
                  pl.BlockSpec((1, cout), lambda i: (0, 0))],
        out_specs=pl.BlockSpec((1, hp, wp, cout), lambda i: (i, 0, 0, 0)),
        compiler_params=pltpu.CompilerParams(
            dimension_semantics=("parallel",),
            vmem_limit_bytes=_VMEM_LIMIT),
    )(y1.reshape(n, h, w, cout), st1, g1.reshape(1, cout),
      be1.reshape(1, cout))

    pv = p.reshape(n, hp // 2, 2, wp // 2, 2 * cout)
    xv = x_nhwc.reshape(n, h // 2, 2, w // 2, 2 * cin)

    def _phase(r, s):
        return pl.BlockSpec((1, hp // 2, 1, wp // 2, cout),
                            lambda i, _r=r, _s=s: (i, 0, _r, 0, _s))

    y2, st2, yd, std = pl.pallas_call(
        functools.partial(_k2b_conv2_ds, ho=ho, wo=wo, wo_pad=wo_pad),
        out_shape=(jax.ShapeDtypeStruct((sp, n * cout), bf),
                   jax.ShapeDtypeStruct((n, 2, cout), jnp.float32),
                   jax.ShapeDtypeStruct((sp, n * c4), bf),
                   jax.ShapeDtypeStruct((n, 2, c4), jnp.float32)),
        grid=(n,),
        in_specs=[_phase(0, 0), _phase(1, 0), _phase(0, 1), _phase(1, 1),
                  pl.BlockSpec((1, h // 2, 1, w // 2, cin),
                               lambda i: (i, 0, 0, 0, 0)),
                  pl.BlockSpec((9 * cout, cout), lambda i: (0, 0)),
                  pl.BlockSpec((1, cout), lambda i: (0, 0)),
                  pl.BlockSpec((cin, c4), lambda i: (0, 0)),
                  pl.BlockSpec((1, c4), lambda i: (0, 0))],
        out_specs=(pl.BlockSpec((sp, cout), lambda i: (0, i)),
                   pl.BlockSpec((1, 2, cout), lambda i: (i, 0, 0)),
                   pl.BlockSpec((sp, c4), lambda i: (0, i)),
                   pl.BlockSpec((1, 2, c4), lambda i: (i, 0, 0))),
        compiler_params=pltpu.CompilerParams(
            dimension_semantics=("parallel",),
            vmem_limit_bytes=_VMEM_LIMIT),
    )(pv, pv, pv, pv, xv, w2.reshape(9 * cout, cout), b2.reshape(1, cout),
      wd, bd.reshape(1, c4))

    m2 = n * sp

    tm = sp
    nt = m2 // tm
    y3, st3 = pl.pallas_call(
        functools.partial(_k3_conv3, m2=m2),
        out_shape=(jax.ShapeDtypeStruct((m2, c4), bf),
                   jax.ShapeDtypeStruct((nt, 2, c4), jnp.float32)),
        grid=(nt,),
        in_specs=[pl.BlockSpec((tm, cout), lambda i: (i, 0)),
                  pl.BlockSpec((n, 2, cout), lambda i: (0, 0, 0)),
                  pl.BlockSpec((1, cout), lambda i: (0, 0)),
                  pl.BlockSpec((1, cout), lambda i: (0, 0)),
                  pl.BlockSpec((cout, c4), lambda i: (0, 0)),
                  pl.BlockSpec((1, c4), lambda i: (0, 0))],
        out_specs=(pl.BlockSpec((tm, c4), lambda i: (i, 0)),
                   pl.BlockSpec((1, 2, c4), lambda i: (i, 0, 0))),
        compiler_params=pltpu.CompilerParams(
            dimension_semantics=("parallel",),
            vmem_limit_bytes=_VMEM_LIMIT),
    )(y2.reshape(m2, cout), st2, g2.reshape(1, cout), be2.reshape(1, cout),
      w3, b3.reshape(1, c4))

    out = pl.pallas_call(
        functools.partial(_k4_residual, m2=m2),
        out_shape=jax.ShapeDtypeStruct((m2, c4), jnp.float32),
        grid=(nt,),
        in_specs=[pl.BlockSpec((tm, c4), lambda i: (i, 0)),
                  pl.BlockSpec((nt, 2, c4), lambda i: (0, 0, 0)),
                  pl.BlockSpec((1, c4), lambda i: (0, 0)),
                  pl.BlockSpec((1, c4), lambda i: (0, 0)),
                  pl.BlockSpec((tm, c4), lambda i: (i, 0)),
                  pl.BlockSpec((n, 2, c4), lambda i: (0, 0, 0)),
                  pl.BlockSpec((1, c4), lambda i: (0, 0)),
                  pl.BlockSpec((1, c4), lambda i: (0, 0))],
        out_specs=pl.BlockSpec((tm, c4), lambda i: (i, 0)),
        compiler_params=pltpu.CompilerParams(
            dimension_semantics=("parallel",),
            vmem_limit_bytes=_VMEM_LIMIT),
    )(y3, st3, g3.reshape(1, c4), be3.reshape(1, c4),
      yd.reshape(m2, c4), std, gd.reshape(1, c4), bed.reshape(1, c4))

    return jnp.transpose(out.reshape(ho, wo, n, c4), (2, 3, 0, 1))


# speedup vs baseline: 13.8990x; 1.9083x over previous
import functools

import jax
import jax.numpy as jnp
from jax.experimental import pallas as pl
from jax.experimental.pallas import tpu as pltpu

_VMEM_LIMIT = 48 * 1024 * 1024
_EPS = 1e-5


def _round_up(a, b):
    return (a + b - 1) // b * b


def _scale_shift(st_ref, g_ref, be_ref, m):
    st = jnp.sum(st_ref[...], axis=0)
    mean = st[0:1] / m
    var = jnp.maximum(st[1:2] / m - mean * mean, 0.0)
    scale = g_ref[...] / jnp.sqrt(var + _EPS)
    shift = be_ref[...] - mean * scale
    return scale, shift


def _k1_conv1_ds(x_ref, w1_ref, b1_ref, wd_ref, bd_ref,
                 y1_ref, st1_ref, yd_ref, std_ref, *, h, w, ho, wo):
    xb = x_ref[0].astype(jnp.bfloat16)
    cin = xb.shape[1]
    y1 = jnp.dot(xb, w1_ref[...].astype(jnp.bfloat16),
                 preferred_element_type=jnp.float32)
    y1 = y1 + b1_ref[...]
    y1_ref[0] = y1.astype(jnp.bfloat16)
    st1_ref[0, 0:1, :] = jnp.sum(y1, axis=0, keepdims=True)
    st1_ref[0, 1:2, :] = jnp.sum(y1 * y1, axis=0, keepdims=True)

    xe = xb.reshape(h // 2, 2, w, cin)[:, 0]
    xe = xe.reshape(ho, wo, 2 * cin)
    xe = xe.reshape(ho * wo, 2 * cin)
    wdx = jnp.pad(wd_ref[...].astype(jnp.bfloat16), ((0, cin), (0, 0)))
    yd = jnp.dot(xe, wdx, preferred_element_type=jnp.float32) + bd_ref[...]
    yd_ref[...] = yd.astype(jnp.bfloat16)
    std_ref[0, 0:1, :] = jnp.sum(yd, axis=0, keepdims=True)
    std_ref[0, 1:2, :] = jnp.sum(yd * yd, axis=0, keepdims=True)


def _k2_conv2(y1_ref, st1_ref, g1_ref, be1_ref, w2_ref, b2_ref,
              y2_ref, st2_ref, *, m1, h, w, ho, wo, wo_pad, hp, wp):
    c = w2_ref.shape[1]
    s1, h1 = _scale_shift(st1_ref, g1_ref, be1_ref, m1)
    a = y1_ref[0].astype(jnp.float32) * s1 + h1
    a = jnp.where(a >= 0, a, 0.02 * a).astype(jnp.bfloat16)
    ap = jnp.pad(a, ((1, hp - h - 1), (1, wp - w - 1), (0, 0)))
    hs = ap.reshape(hp // 2, 2, wp, c)
    merged = [hs[:, r].reshape(hp // 2, wp // 2, 2 * c) for r in range(2)]
    wins = []
    for di in range(3):
        for dj in range(3):
            r, s = di % 2, dj % 2
            v = merged[r][:, :, s * c:(s + 1) * c]
            if dj == 2:
                v = jnp.roll(v, -1, axis=1)
            wins.append(v[di // 2:di // 2 + ho].reshape(ho * (wp // 2), c))
    xw = jnp.concatenate(wins, axis=1)
    y2 = jnp.dot(xw, w2_ref[...].astype(jnp.bfloat16),
                 preferred_element_type=jnp.float32) + b2_ref[...]
    y2 = y2.reshape(ho, wp // 2, c)[:, :wo, :].reshape(ho * wo, c)
    y2_ref[...] = y2.astype(jnp.bfloat16)
    st2_ref[0, 0:1, :] = jnp.sum(y2, axis=0, keepdims=True)
    st2_ref[0, 1:2, :] = jnp.sum(y2 * y2, axis=0, keepdims=True)


def _k3_conv3(y2_ref, st2_ref, g2_ref, be2_ref, w3_ref, b3_ref,
              y3_ref, st3_ref, *, m2):
    s2, h2 = _scale_shift(st2_ref, g2_ref, be2_ref, m2)
    t = y2_ref[...].astype(jnp.float32) * s2 + h2
    a2 = jnp.where(t >= 0, t, 0.02 * t).astype(jnp.bfloat16)
    y3 = jnp.dot(a2, w3_ref[...].astype(jnp.bfloat16),
                 preferred_element_type=jnp.float32) + b3_ref[...]
    y3_ref[...] = y3.astype(jnp.bfloat16)
    st3_ref[0, 0:1, :] = jnp.sum(y3, axis=0, keepdims=True)
    st3_ref[0, 1:2, :] = jnp.sum(y3 * y3, axis=0, keepdims=True)


def _k4_residual(y3_ref, st3_ref, g3_ref, be3_ref, yd_ref, std_ref,
                 gd_ref, bed_ref, o_ref, *, m2):
    s3, h3 = _scale_shift(st3_ref, g3_ref, be3_ref, m2)
    sd, hd = _scale_shift(std_ref, gd_ref, bed_ref, m2)
    z = y3_ref[...].astype(jnp.float32) * s3 + h3
    sk = yd_ref[...].astype(jnp.float32) * sd + hd
    y = z + sk
    o_ref[...] = jnp.where(y >= 0, y, 0.01 * y)


def kernel(x, w1, b1, g1, be1, w2, b2, g2, be2, w3, b3, g3, be3,
           wd, bd, gd, bed):
    n, cin, h, w = x.shape
    cout = w1.shape[1]
    c4 = w3.shape[1]
    ho = (h + 2 - 3) // 2 + 1
    wo = (w + 2 - 3) // 2 + 1
    wo_pad = _round_up(wo, 8)
    hw = h * w
    sp = ho * wo
    bf = jnp.bfloat16
    hp = _round_up(h + 2, 16)
    wp = _round_up(w + 2, 16)

    x_img = jnp.transpose(x, (0, 2, 3, 1)).reshape(n, hw, cin)

    m1 = n * hw
    y1, st1, yd, std = pl.pallas_call(
        functools.partial(_k1_conv1_ds, h=h, w=w, ho=ho, wo=wo),
        out_shape=(jax.ShapeDtypeStruct((n, hw, cout), bf),
                   jax.ShapeDtypeStruct((n, 2, cout), jnp.float32),
                   jax.ShapeDtypeStruct((sp, n * c4), bf),
                   jax.ShapeDtypeStruct((n, 2, c4), jnp.float32)),
        grid=(n,),
        in_specs=[pl.BlockSpec((1, hw, cin), lambda i: (i, 0, 0)),
                  pl.BlockSpec((cin, cout), lambda i: (0, 0)),
                  pl.BlockSpec((1, cout), lambda i: (0, 0)),
                  pl.BlockSpec((cin, c4), lambda i: (0, 0)),
                  pl.BlockSpec((1, c4), lambda i: (0, 0))],
        out_specs=(pl.BlockSpec((1, hw, cout), lambda i: (i, 0, 0)),
                   pl.BlockSpec((1, 2, cout), lambda i: (i, 0, 0)),
                   pl.BlockSpec((sp, c4), lambda i: (0, i)),
                   pl.BlockSpec((1, 2, c4), lambda i: (i, 0, 0))),
        compiler_params=pltpu.CompilerParams(
            dimension_semantics=("parallel",),
            vmem_limit_bytes=_VMEM_LIMIT),
    )(x_img, w1, b1.reshape(1, cout), wd, bd.reshape(1, c4))

    y2, st2 = pl.pallas_call(
        functools.partial(_k2_conv2, m1=m1, h=h, w=w, ho=ho, wo=wo,
                          wo_pad=wo_pad, hp=hp, wp=wp),
        out_shape=(jax.ShapeDtypeStruct((sp, n * cout), bf),
                   jax.ShapeDtypeStruct((n, 2, cout), jnp.float32)),
        grid=(n,),
        in_specs=[pl.BlockSpec((1, h, w, cout), lambda i: (i, 0, 0, 0)),
                  pl.BlockSpec((n, 2, cout), lambda i: (0, 0, 0)),
                  pl.BlockSpec((1, cout), lambda i: (0, 0)),
                  pl.BlockSpec((1, cout), lambda i: (0, 0)),
                  pl.BlockSpec((9 * cout, cout), lambda i: (0, 0)),
                  pl.BlockSpec((1, cout), lambda i: (0, 0))],
        out_specs=(pl.BlockSpec((sp, cout), lambda i: (0, i)),
                   pl.BlockSpec((1, 2, cout), lambda i: (i, 0, 0))),
        compiler_params=pltpu.CompilerParams(
            dimension_semantics=("parallel",),
            vmem_limit_bytes=_VMEM_LIMIT),
    )(y1.reshape(n, h, w, cout), st1, g1.reshape(1, cout),
      be1.reshape(1, cout), w2.reshape(9 * cout, cout), b2.reshape(1, cout))

    m2 = n * sp

    tm = sp
    nt = m2 // tm
    y3, st3 = pl.pallas_call(
        functools.partial(_k3_conv3, m2=m2),
        out_shape=(jax.ShapeDtypeStruct((m2, c4), bf),
                   jax.ShapeDtypeStruct((nt, 2, c4), jnp.float32)),
        grid=(nt,),
        in_specs=[pl.BlockSpec((tm, cout), lambda i: (i, 0)),
                  pl.BlockSpec((n, 2, cout), lambda i: (0, 0, 0)),
                  pl.BlockSpec((1, cout), lambda i: (0, 0)),
                  pl.BlockSpec((1, cout), lambda i: (0, 0)),
                  pl.BlockSpec((cout, c4), lambda i: (0, 0)),
                  pl.BlockSpec((1, c4), lambda i: (0, 0))],
        out_specs=(pl.BlockSpec((tm, c4), lambda i: (i, 0)),
                   pl.BlockSpec((1, 2, c4), lambda i: (i, 0, 0))),
        compiler_params=pltpu.CompilerParams(
            dimension_semantics=("parallel",),
            vmem_limit_bytes=_VMEM_LIMIT),
    )(y2.reshape(m2, cout), st2, g2.reshape(1, cout), be2.reshape(1, cout),
      w3, b3.reshape(1, c4))

    out = pl.pallas_call(
        functools.partial(_k4_residual, m2=m2),
        out_shape=jax.ShapeDtypeStruct((m2, c4), jnp.float32),
        grid=(nt,),
        in_specs=[pl.BlockSpec((tm, c4), lambda i: (i, 0)),
                  pl.BlockSpec((nt, 2, c4), lambda i: (0, 0, 0)),
                  pl.BlockSpec((1, c4), lambda i: (0, 0)),
                  pl.BlockSpec((1, c4), lambda i: (0, 0)),
                  pl.BlockSpec((tm, c4), lambda i: (i, 0)),
                  pl.BlockSpec((n, 2, c4), lambda i: (0, 0, 0)),
                  pl.BlockSpec((1, c4), lambda i: (0, 0)),
                  pl.BlockSpec((1, c4), lambda i: (0, 0))],
        out_specs=pl.BlockSpec((tm, c4), lambda i: (i, 0)),
        compiler_params=pltpu.CompilerParams(
            dimension_semantics=("parallel",),
            vmem_limit_bytes=_VMEM_LIMIT),
    )(y3, st3, g3.reshape(1, c4), be3.reshape(1, c4),
      yd.reshape(m2, c4), std, gd.reshape(1, c4), bed.reshape(1, c4))

    return jnp.transpose(out.reshape(ho, wo, n, c4), (2, 3, 0, 1))


# speedup vs baseline: 17.1637x; 1.2349x over previous
import functools

import jax
import jax.numpy as jnp
from jax.experimental import pallas as pl
from jax.experimental.pallas import tpu as pltpu

_VMEM_LIMIT = 48 * 1024 * 1024
_EPS = 1e-5


def _round_up(a, b):
    return (a + b - 1) // b * b


def _scale_shift(st_ref, g_ref, be_ref, m):
    st = jnp.sum(st_ref[...], axis=0)
    mean = st[0:1] / m
    var = jnp.maximum(st[1:2] / m - mean * mean, 0.0)
    scale = g_ref[...] / jnp.sqrt(var + _EPS)
    shift = be_ref[...] - mean * scale
    return scale, shift


def _k1_conv1_ds(x_ref, w1_ref, b1_ref, wd_ref, bd_ref,
                 y1_ref, st1_ref, yd_ref, std_ref, *, h, w, ho, wo):
    xb = x_ref[0].astype(jnp.bfloat16)
    cin = xb.shape[1]
    y1 = jnp.dot(xb, w1_ref[...].astype(jnp.bfloat16),
                 preferred_element_type=jnp.float32)
    y1 = y1 + b1_ref[...]
    y1_ref[0] = y1.astype(jnp.bfloat16)
    st1_ref[0, 0:1, :] = jnp.sum(y1, axis=0, keepdims=True)
    st1_ref[0, 1:2, :] = jnp.sum(y1 * y1, axis=0, keepdims=True)

    xe = xb.reshape(h // 2, 2, w, cin)[:, 0]
    xe = xe.reshape(ho, wo, 2 * cin)
    xe = xe.reshape(ho * wo, 2 * cin)
    wdx = jnp.pad(wd_ref[...].astype(jnp.bfloat16), ((0, cin), (0, 0)))
    yd = jnp.dot(xe, wdx, preferred_element_type=jnp.float32) + bd_ref[...]
    yd_ref[...] = yd.astype(jnp.bfloat16)
    std_ref[0, 0:1, :] = jnp.sum(yd, axis=0, keepdims=True)
    std_ref[0, 1:2, :] = jnp.sum(yd * yd, axis=0, keepdims=True)


def _k2_conv2(y1_ref, st1_ref, g1_ref, be1_ref, w2_ref, b2_ref,
              y2_ref, st2_ref, *, m1, h, w, ho, wo, wo_pad, hp, wp):
    c = w2_ref.shape[1]
    s1, h1 = _scale_shift(st1_ref, g1_ref, be1_ref, m1)
    a = y1_ref[0].astype(jnp.float32) * s1 + h1
    a = jnp.where(a >= 0, a, 0.02 * a).astype(jnp.bfloat16)
    ap = jnp.pad(a, ((1, hp - h - 1), (1, wp - w - 1), (0, 0)))
    hs = ap.reshape(hp // 2, 2, wp, c)
    merged = [hs[:, r].reshape(hp // 2, wp // 2, 2 * c) for r in range(2)]
    wins = []
    for di in range(3):
        for dj in range(3):
            r, s = di % 2, dj % 2
            v = merged[r][:, :, s * c:(s + 1) * c]
            if dj == 2:
                v = jnp.roll(v, -1, axis=1)
            wins.append(v[di // 2:di // 2 + ho].reshape(ho * (wp // 2), c))
    xw = jnp.concatenate(wins, axis=1)
    y2 = jnp.dot(xw, w2_ref[...].astype(jnp.bfloat16),
                 preferred_element_type=jnp.float32) + b2_ref[...]
    y2 = y2.reshape(ho, wp // 2, c)[:, :wo, :].reshape(ho * wo, c)
    y2_ref[...] = y2.astype(jnp.bfloat16)
    st2_ref[0, 0:1, :] = jnp.sum(y2, axis=0, keepdims=True)
    st2_ref[0, 1:2, :] = jnp.sum(y2 * y2, axis=0, keepdims=True)


def _k3_conv3(y2_ref, st2_ref, g2_ref, be2_ref, w3_ref, b3_ref,
              y3_ref, st3_ref, *, m2):
    s2, h2 = _scale_shift(st2_ref, g2_ref, be2_ref, m2)
    t = y2_ref[...].astype(jnp.float32) * s2 + h2
    a2 = jnp.where(t >= 0, t, 0.02 * t).astype(jnp.bfloat16)
    y3 = jnp.dot(a2, w3_ref[...].astype(jnp.bfloat16),
                 preferred_element_type=jnp.float32) + b3_ref[...]
    y3_ref[...] = y3.astype(jnp.bfloat16)
    st3_ref[0, 0:1, :] = jnp.sum(y3, axis=0, keepdims=True)
    st3_ref[0, 1:2, :] = jnp.sum(y3 * y3, axis=0, keepdims=True)


def _k4_residual(y3_ref, st3_ref, g3_ref, be3_ref, yd_ref, std_ref,
                 gd_ref, bed_ref, o_ref, *, m2):
    s3, h3 = _scale_shift(st3_ref, g3_ref, be3_ref, m2)
    sd, hd = _scale_shift(std_ref, gd_ref, bed_ref, m2)
    tm, c4 = o_ref.shape
    ydv = yd_ref[...].reshape(tm, c4)
    z = y3_ref[...].astype(jnp.float32) * s3 + h3
    sk = ydv.astype(jnp.float32) * sd + hd
    y = z + sk
    o_ref[...] = jnp.where(y >= 0, y, 0.01 * y)


def kernel(x, w1, b1, g1, be1, w2, b2, g2, be2, w3, b3, g3, be3,
           wd, bd, gd, bed):
    n, cin, h, w = x.shape
    cout = w1.shape[1]
    c4 = w3.shape[1]
    ho = (h + 2 - 3) // 2 + 1
    wo = (w + 2 - 3) // 2 + 1
    wo_pad = _round_up(wo, 8)
    hw = h * w
    sp = ho * wo
    bf = jnp.bfloat16
    hp = _round_up(h + 2, 16)
    wp = _round_up(w + 2, 16)

    x_img = jnp.transpose(x, (0, 2, 3, 1)).reshape(n, hw, cin)

    m1 = n * hw
    y1, st1, yd, std = pl.pallas_call(
        functools.partial(_k1_conv1_ds, h=h, w=w, ho=ho, wo=wo),
        out_shape=(jax.ShapeDtypeStruct((n, hw, cout), bf),
                   jax.ShapeDtypeStruct((n, 2, cout), jnp.float32),
                   jax.ShapeDtypeStruct((sp, n * c4), bf),
                   jax.ShapeDtypeStruct((n, 2, c4), jnp.float32)),
        grid=(n,),
        in_specs=[pl.BlockSpec((1, hw, cin), lambda i: (i, 0, 0)),
                  pl.BlockSpec((cin, cout), lambda i: (0, 0)),
                  pl.BlockSpec((1, cout), lambda i: (0, 0)),
                  pl.BlockSpec((cin, c4), lambda i: (0, 0)),
                  pl.BlockSpec((1, c4), lambda i: (0, 0))],
        out_specs=(pl.BlockSpec((1, hw, cout), lambda i: (i, 0, 0)),
                   pl.BlockSpec((1, 2, cout), lambda i: (i, 0, 0)),
                   pl.BlockSpec((sp, c4), lambda i: (0, i)),
                   pl.BlockSpec((1, 2, c4), lambda i: (i, 0, 0))),
        compiler_params=pltpu.CompilerParams(
            dimension_semantics=("parallel",),
            vmem_limit_bytes=_VMEM_LIMIT),
    )(x_img, w1, b1.reshape(1, cout), wd, bd.reshape(1, c4))

    y2, st2 = pl.pallas_call(
        functools.partial(_k2_conv2, m1=m1, h=h, w=w, ho=ho, wo=wo,
                          wo_pad=wo_pad, hp=hp, wp=wp),
        out_shape=(jax.ShapeDtypeStruct((sp, n * cout), bf),
                   jax.ShapeDtypeStruct((n, 2, cout), jnp.float32)),
        grid=(n,),
        in_specs=[pl.BlockSpec((1, h, w, cout), lambda i: (i, 0, 0, 0)),
                  pl.BlockSpec((n, 2, cout), lambda i: (0, 0, 0)),
                  pl.BlockSpec((1, cout), lambda i: (0, 0)),
                  pl.BlockSpec((1, cout), lambda i: (0, 0)),
                  pl.BlockSpec((9 * cout, cout), lambda i: (0, 0)),
                  pl.BlockSpec((1, cout), lambda i: (0, 0))],
        out_specs=(pl.BlockSpec((sp, cout), lambda i: (0, i)),
                   pl.BlockSpec((1, 2, cout), lambda i: (i, 0, 0))),
        compiler_params=pltpu.CompilerParams(
            dimension_semantics=("parallel",),
            vmem_limit_bytes=_VMEM_LIMIT),
    )(y1.reshape(n, h, w, cout), st1, g1.reshape(1, cout),
      be1.reshape(1, cout), w2.reshape(9 * cout, cout), b2.reshape(1, cout))

    m2 = n * sp

    tm = sp
    nt = m2 // tm
    y3, st3 = pl.pallas_call(
        functools.partial(_k3_conv3, m2=m2),
        out_shape=(jax.ShapeDtypeStruct((m2, c4), bf),
                   jax.ShapeDtypeStruct((nt, 2, c4), jnp.float32)),
        grid=(nt,),
        in_specs=[pl.BlockSpec((tm, cout), lambda i: (i, 0)),
                  pl.BlockSpec((n, 2, cout), lambda i: (0, 0, 0)),
                  pl.BlockSpec((1, cout), lambda i: (0, 0)),
                  pl.BlockSpec((1, cout), lambda i: (0, 0)),
                  pl.BlockSpec((cout, c4), lambda i: (0, 0)),
                  pl.BlockSpec((1, c4), lambda i: (0, 0))],
        out_specs=(pl.BlockSpec((tm, c4), lambda i: (i, 0)),
                   pl.BlockSpec((1, 2, c4), lambda i: (i, 0, 0))),
        compiler_params=pltpu.CompilerParams(
            dimension_semantics=("parallel",),
            vmem_limit_bytes=_VMEM_LIMIT),
    )(y2.reshape(m2, cout), st2, g2.reshape(1, cout), be2.reshape(1, cout),
      w3, b3.reshape(1, c4))

    nt4 = 7 if (sp % 7 == 0 and (sp // 7) % 8 == 0) else 1
    tm4 = m2 // nt4
    out = pl.pallas_call(
        functools.partial(_k4_residual, m2=m2),
        out_shape=jax.ShapeDtypeStruct((m2, c4), jnp.float32),
        grid=(nt4,),
        in_specs=[pl.BlockSpec((tm4, c4), lambda i: (i, 0)),
                  pl.BlockSpec((nt, 2, c4), lambda i: (0, 0, 0)),
                  pl.BlockSpec((1, c4), lambda i: (0, 0)),
                  pl.BlockSpec((1, c4), lambda i: (0, 0)),
                  pl.BlockSpec((tm4 // n, n * c4), lambda i: (i, 0)),
                  pl.BlockSpec((n, 2, c4), lambda i: (0, 0, 0)),
                  pl.BlockSpec((1, c4), lambda i: (0, 0)),
                  pl.BlockSpec((1, c4), lambda i: (0, 0))],
        out_specs=pl.BlockSpec((tm4, c4), lambda i: (i, 0)),
        compiler_params=pltpu.CompilerParams(
            dimension_semantics=("parallel",),
            vmem_limit_bytes=_VMEM_LIMIT),
    )(y3, st3, g3.reshape(1, c4), be3.reshape(1, c4),
      yd, std, gd.reshape(1, c4), bed.reshape(1, c4))

    return jnp.transpose(out.reshape(ho, wo, n, c4), (2, 3, 0, 1))
